# Initial kernel scaffold; baseline (speedup 1.0000x reference)
import jax
import jax.numpy as jnp
from jax.experimental import pallas as pl


def kernel(input_ids, tables):
    raise NotImplementedError("write your pallas kernel here")


# speedup vs baseline: 5.6525x; 5.6525x over previous
import functools

import jax
import jax.numpy as jnp
from jax import lax
from jax.experimental import pallas as pl
from jax.experimental.pallas import tpu as pltpu
from jax.experimental.pallas import tpu_sc as plsc

NUM_QUANT = 8
NUM_EMB = 1034
EMB_DIM = 1024
B = 4
T = 4096

NC = 2
NS = 16
NW = NC * NS
LANES = 16

TOKENS = B * T
TOK_PER_W = TOKENS // NW
C = 4
STEPS = TOK_PER_W // C
ROWS_PER_STEP = C * NUM_QUANT
HCHUNKS = EMB_DIM // LANES


def _sc_body(ftab_hbm, idx_hbm, out_hbm, idx_v, rows_v, acc_v, sem0, sem1):
    wid = lax.axis_index("s") * NC + lax.axis_index("c")
    base = wid * TOK_PER_W

    pltpu.sync_copy(idx_hbm.at[wid], idx_v)

    sems = (sem0, sem1)

    def gather_start(g, b):
        pltpu.make_async_copy(
            ftab_hbm.at[idx_v.at[g]], rows_v.at[b], sems[b]
        ).start()

    def gather_wait(g, b):
        pltpu.make_async_copy(
            ftab_hbm.at[idx_v.at[g]], rows_v.at[b], sems[b]
        ).wait()

    gather_start(0, 0)
    gather_start(1, 1)

    def step(g, b):
        gather_wait(g, b)
        def h_body(h, _):
            col = pl.ds(h * LANES, LANES)
            for c in range(C):
                acc = rows_v[b, c * NUM_QUANT, col]
                for q in range(1, NUM_QUANT):
                    acc = acc + rows_v[b, c * NUM_QUANT + q, col]
                acc_v[c, col] = acc
            return 0
        lax.fori_loop(0, HCHUNKS, h_body, 0, unroll=2)
        pltpu.sync_copy(acc_v, out_hbm.at[pl.ds(base + g * C, C)])
        @pl.when(g + 2 < STEPS)
        def _():
            gather_start(g + 2, b)

    def pair(p, _):
        step(p * 2, 0)
        step(p * 2 + 1, 1)
        return 0

    lax.fori_loop(0, STEPS // 2, pair, 0)


@jax.jit
def _multi_embedding_sum(flat_tables, idx):
    mesh = plsc.VectorSubcoreMesh(core_axis_name="c", subcore_axis_name="s")
    scratch = [
        pltpu.VMEM((STEPS, ROWS_PER_STEP), jnp.int32),
        pltpu.VMEM((2, ROWS_PER_STEP, EMB_DIM), jnp.float32),
        pltpu.VMEM((C, EMB_DIM), jnp.float32),
        pltpu.SemaphoreType.DMA,
        pltpu.SemaphoreType.DMA,
    ]
    run = pl.kernel(
        _sc_body,
        out_type=jax.ShapeDtypeStruct((TOKENS, EMB_DIM), jnp.float32),
        mesh=mesh,
        scratch_types=scratch,
    )
    return run(flat_tables, idx)


def kernel(input_ids, tables):
    flat_tables = tables.reshape(NUM_QUANT * NUM_EMB, EMB_DIM)
    offs = (jnp.arange(NUM_QUANT, dtype=jnp.int32) * NUM_EMB)[:, None]
    flat_ids = input_ids.reshape(NUM_QUANT, TOKENS) + offs
    idx = flat_ids.T.reshape(NW, STEPS, ROWS_PER_STEP)
    out = _multi_embedding_sum(flat_tables, idx)
    return out.reshape(B, T, EMB_DIM)
